# baseline (speedup 1.0000x reference)
import functools

import jax
import jax.numpy as jnp
from jax import lax
from jax.experimental import pallas as pl
from jax.experimental.pallas import tpu as pltpu
from jax.experimental.pallas import tpu_sc as plsc

B = 4
S = 2048
D = 768
N = B * S
NC = 2
NS = 16
NW = NC * NS
POS_PER_W = S // NW
H = 2
CH = POS_PER_W // H
NCHUNK = H * B
LANES = 16
D_VECS = D // LANES
ST = 8
DT = 128
S_TILES_PER_HALF = CH // ST
D_TILES = D // DT


def _pe_table():
    pos = jnp.arange(S, dtype=jnp.float32)[:, None]
    i = jnp.arange(D // 2, dtype=jnp.float32)[None, :]
    angles = pos / jnp.power(10000.0, 2.0 * i / D)
    return jnp.stack([jnp.sin(angles), jnp.cos(angles)], axis=-1).reshape(S, D)


_mesh = plsc.VectorSubcoreMesh(core_axis_name="c", subcore_axis_name="s")


@functools.partial(
    pl.kernel,
    mesh=_mesh,
    out_type=jax.ShapeDtypeStruct((N, D), jnp.float32),
    scratch_types=[
        pltpu.VMEM((H, B, CH), jnp.int32),
        pltpu.VMEM((S_TILES_PER_HALF, D_TILES, ST, DT), jnp.float32),
        pltpu.VMEM((2, CH, D), jnp.float32),
        pltpu.SemaphoreType.DMA,
        pltpu.SemaphoreType.DMA,
        pltpu.SemaphoreType.DMA,
        pltpu.SemaphoreType.DMA,
        pltpu.SemaphoreType.DMA,
    ],
)
def _embed_pe(idx_hbm, table_hbm, pe_hbm, out_hbm,
              idx_v, pe_v, rows_v, g_sem0, g_sem1, s_sem0, s_sem1, i_sem):
    wid = lax.axis_index("s") * NC + lax.axis_index("c")
    base = wid * POS_PER_W

    idx_copies = []
    for h in range(H):
        for b in range(B):
            idx_copies.append(pltpu.async_copy(
                idx_hbm.at[b, pl.ds(base + h * CH, CH)],
                idx_v.at[h, b], i_sem))
    for cp in idx_copies:
        cp.wait()

    g_sems = (g_sem0, g_sem1)
    s_sems = (s_sem0, s_sem1)
    chunks = [(h, b) for h in range(H) for b in range(B)]
    gathers = [None] * NCHUNK
    stores = [None] * NCHUNK

    def issue_gather(k):
        h, b = chunks[k]
        p = k % 2
        gathers[k] = pltpu.async_copy(
            table_hbm.at[idx_v.at[h, b]], rows_v.at[p], g_sems[p])

    issue_gather(0)
    pe_loaded = -1
    for k in range(NCHUNK):
        h, b = chunks[k]
        p = k % 2
        if h != pe_loaded:
            stile = wid * (POS_PER_W // ST) + h * S_TILES_PER_HALF
            pltpu.sync_copy(
                pe_hbm.at[pl.ds(stile, S_TILES_PER_HALF)], pe_v)
            pe_loaded = h
        gathers[k].wait()
        if k + 1 < NCHUNK:
            if k >= 1 and stores[k - 1] is not None:
                stores[k - 1].wait()
            issue_gather(k + 1)

        def row_add(i, carry):
            st = i // ST
            si = lax.rem(i, ST)
            for j in range(D_VECS):
                sl = pl.ds(j * LANES, LANES)
                dt = j // (DT // LANES)
                off = pl.ds((j % (DT // LANES)) * LANES, LANES)
                rows_v[p, i, sl] = rows_v[p, i, sl] + pe_v[st, dt, si, off]
            return carry

        lax.fori_loop(0, CH, row_add, 0)
        stores[k] = pltpu.async_copy(
            rows_v.at[p], out_hbm.at[pl.ds(b * S + base + h * CH, CH)],
            s_sems[p])
    stores[NCHUNK - 2].wait()
    stores[NCHUNK - 1].wait()


def kernel(input, table):
    pe = (_pe_table()
          .reshape(S // ST, ST, D_TILES, DT)
          .transpose(0, 2, 1, 3))
    out = _embed_pe(input.astype(jnp.int32), table, pe)
    return out.reshape(B, S, D)


# speedup vs baseline: 3.2721x; 3.2721x over previous
import functools

import jax
import jax.numpy as jnp
import numpy as np
from jax import lax
from jax.experimental import pallas as pl
from jax.experimental.pallas import tpu as pltpu
from jax.experimental.pallas import tpu_sc as plsc

B = 4
S = 2048
D = 768
N = B * S
NC = 2
NS = 16
NW = NC * NS
POS_PER_W = S // NW
H = 2
CH = POS_PER_W // H
NCHUNK = H * B
LANES = 16
D_VECS = D // LANES
ST = 8
DT = 128
S_TILES_PER_HALF = CH // ST
D_TILES = D // DT


def _pe_table_np():
    pos = np.arange(S, dtype=np.float32)[:, None]
    i = np.arange(D // 2, dtype=np.float32)[None, :]
    angles = (pos / np.power(10000.0, 2.0 * i / D)).astype(np.float32)
    pe = np.stack([np.sin(angles), np.cos(angles)], axis=-1)
    return pe.reshape(S, D).astype(np.float32)


_PE4 = np.ascontiguousarray(
    _pe_table_np().reshape(S // 8, 8, D // 128, 128).transpose(0, 2, 1, 3))


_mesh = plsc.VectorSubcoreMesh(core_axis_name="c", subcore_axis_name="s")


@functools.partial(
    pl.kernel,
    mesh=_mesh,
    out_type=jax.ShapeDtypeStruct((N, D), jnp.float32),
    scratch_types=[
        pltpu.VMEM((H, B, CH), jnp.int32),
        pltpu.VMEM((S_TILES_PER_HALF, D_TILES, ST, DT), jnp.float32),
        pltpu.VMEM((2, CH, D), jnp.float32),
        pltpu.SemaphoreType.DMA,
        pltpu.SemaphoreType.DMA,
        pltpu.SemaphoreType.DMA,
        pltpu.SemaphoreType.DMA,
        pltpu.SemaphoreType.DMA,
    ],
)
def _embed_pe(idx_hbm, table_hbm, pe_hbm, out_hbm,
              idx_v, pe_v, rows_v, g_sem0, g_sem1, s_sem0, s_sem1, i_sem):
    wid = lax.axis_index("s") * NC + lax.axis_index("c")
    base = wid * POS_PER_W

    idx_copies = []
    for h in range(H):
        for b in range(B):
            idx_copies.append(pltpu.async_copy(
                idx_hbm.at[b, pl.ds(base + h * CH, CH)],
                idx_v.at[h, b], i_sem))
    for cp in idx_copies:
        cp.wait()

    g_sems = (g_sem0, g_sem1)
    s_sems = (s_sem0, s_sem1)
    chunks = [(h, b) for h in range(H) for b in range(B)]
    gathers = [None] * NCHUNK
    stores = [None] * NCHUNK

    def issue_gather(k):
        h, b = chunks[k]
        p = k % 2
        gathers[k] = pltpu.async_copy(
            table_hbm.at[idx_v.at[h, b]], rows_v.at[p], g_sems[p])

    issue_gather(0)
    pe_loaded = -1
    for k in range(NCHUNK):
        h, b = chunks[k]
        p = k % 2
        if h != pe_loaded:
            stile = wid * (POS_PER_W // ST) + h * S_TILES_PER_HALF
            pltpu.sync_copy(
                pe_hbm.at[pl.ds(stile, S_TILES_PER_HALF)], pe_v)
            pe_loaded = h
        gathers[k].wait()
        if k + 1 < NCHUNK:
            if k >= 1 and stores[k - 1] is not None:
                stores[k - 1].wait()
            issue_gather(k + 1)

        def row_add(i, carry):
            st = i // ST
            si = lax.rem(i, ST)
            for j in range(D_VECS):
                sl = pl.ds(j * LANES, LANES)
                dt = j // (DT // LANES)
                off = pl.ds((j % (DT // LANES)) * LANES, LANES)
                rows_v[p, i, sl] = rows_v[p, i, sl] + pe_v[st, dt, si, off]
            return carry

        lax.fori_loop(0, CH, row_add, 0)
        stores[k] = pltpu.async_copy(
            rows_v.at[p], out_hbm.at[pl.ds(b * S + base + h * CH, CH)],
            s_sems[p])
    stores[NCHUNK - 2].wait()
    stores[NCHUNK - 1].wait()


def kernel(input, table):
    pe = jnp.asarray(_PE4)
    out = _embed_pe(input.astype(jnp.int32), table, pe)
    return out.reshape(B, S, D)
